# baseline (speedup 1.0000x reference)
import functools

import jax
import jax.numpy as jnp
from jax import lax
from jax.experimental import pallas as pl
from jax.experimental.pallas import tpu as pltpu
from jax.experimental.pallas import tpu_sc as plsc

D = 128
B = 4096
L = 200

NC = 2
NS = 16
NW = NC * NS
RPW = B // NW
LANES = 16
NG = D // 32
C0, C1 = 104, 96
HMASK = -65536

_mesh = plsc.VectorSubcoreMesh(core_axis_name="c", subcore_axis_name="s")


def _split(v):
    lo = plsc.bitcast(jnp.left_shift(v, 16), jnp.float32)
    hi = plsc.bitcast(jnp.bitwise_and(v, HMASK), jnp.float32)
    return lo, hi


def _pool_body(body_hbm, table_hbm, out_hbm, idx_v, buf0, buf1, buf2, buf3,
               out4, e0_v, sem0, sem1, sem2, sem3, outsem):
    wid = lax.axis_index("s") * NC + lax.axis_index("c")
    base = wid * RPW
    pltpu.sync_copy(body_hbm.at[pl.ds(base, RPW)], idx_v)
    pltpu.sync_copy(table_hbm.at[pl.ds(0, 1)], e0_v)

    def _issue(row, buf, sem):
        pltpu.async_copy(table_hbm.at[idx_v.at[row, pl.ds(0, C0)]],
                         buf.at[pl.ds(0, C0)], sem)
        pltpu.async_copy(table_hbm.at[idx_v.at[row, pl.ds(C0, C1)]],
                         buf.at[pl.ds(C0, C1)], sem)

    def _wait(buf, sem):
        pltpu.make_async_copy(table_hbm.at[idx_v.at[0, pl.ds(0, C0)]],
                              buf.at[pl.ds(0, C0)], sem).wait()
        pltpu.make_async_copy(table_hbm.at[idx_v.at[0, pl.ds(C0, C1)]],
                              buf.at[pl.ds(C0, C1)], sem).wait()

    lane = lax.iota(jnp.int32, LANES)
    idx_a = lax.shift_right_logical(lane, 1)
    idx_b = idx_a + 8
    even = jnp.bitwise_and(lane, 1) == 0
    e0 = [_split(e0_v[0, pl.ds(g * LANES, LANES)]) for g in range(NG)]
    zero = jnp.zeros((LANES,), jnp.float32)

    _dnums = lax.GatherDimensionNumbers(offset_dims=(),
                                        collapsed_slice_dims=(0,),
                                        start_index_map=(0,))

    def _take(v, i):
        return lax.gather(v, i[:, None], _dnums, slice_sizes=(1,),
                          mode=lax.GatherScatterMode.PROMISE_IN_BOUNDS)

    def _process(row, buf, slot):
        cnt = zero
        for c in range(12):
            cnt = cnt + jnp.where(idx_v[row, pl.ds(c * 16, 16)] > 0, 1.0, 0.0)
        tail = (idx_v[row, pl.ds(184, 16)] > 0) & (lane >= 8)
        cnt = cnt + jnp.where(tail, 1.0, 0.0)
        nnzf = jnp.broadcast_to(jnp.sum(cnt), (LANES,))
        n0 = float(L) - nnzf
        inv = 1.0 / jnp.maximum(nnzf, 1.0)

        def acc_step(t, accs):
            los, his = accs
            l = t * 4
            for u in range(4):
                for g in range(NG):
                    lo, hi = _split(buf[l + u, pl.ds(g * LANES, LANES)])
                    los = tuple(los[k] + lo if k == g else los[k]
                                for k in range(NG))
                    his = tuple(his[k] + hi if k == g else his[k]
                                for k in range(NG))
            return los, his

        los, his = lax.fori_loop(
            0, L // 4, acc_step,
            (tuple(zero for _ in range(NG)), tuple(zero for _ in range(NG))))
        for g in range(NG):
            lo = (los[g] - n0 * e0[g][0]) * inv
            hi = (his[g] - n0 * e0[g][1]) * inv
            fa = jnp.where(even, _take(lo, idx_a), _take(hi, idx_a))
            fb = jnp.where(even, _take(lo, idx_b), _take(hi, idx_b))
            out4[slot, pl.ds(g * 32, 16)] = fa
            out4[slot, pl.ds(g * 32 + 16, 16)] = fb
        pltpu.async_copy(out4.at[pl.ds(slot, 1)],
                         out_hbm.at[pl.ds(base + row, 1)], outsem)

    bufs = (buf0, buf1, buf2, buf3)
    sems = (sem0, sem1, sem2, sem3)
    for b in range(4):
        _issue(b, bufs[b], sems[b])

    def _wait_store(slot):
        pltpu.make_async_copy(out4.at[pl.ds(slot, 1)],
                              out_hbm.at[pl.ds(base, 1)], outsem).wait()

    def outer(t, carry):
        for b in range(4):
            row = 4 * t + b
            _wait(bufs[b], sems[b])

            @pl.when(row >= 4)
            def _():
                _wait_store(b)

            _process(row, bufs[b], b)

            @pl.when(row + 4 < RPW)
            def _():
                _issue(row + 4, bufs[b], sems[b])

        return carry

    lax.fori_loop(0, RPW // 4, outer, 0)
    for b in range(4):
        _wait_store(b)


_pool = functools.partial(
    pl.kernel,
    out_type=jax.ShapeDtypeStruct((B, D), jnp.float32),
    mesh=_mesh,
    scratch_types=[
        pltpu.VMEM((RPW, L), jnp.int32),
        pltpu.VMEM((L, D // 2), jnp.int32),
        pltpu.VMEM((L, D // 2), jnp.int32),
        pltpu.VMEM((L, D // 2), jnp.int32),
        pltpu.VMEM((L, D // 2), jnp.int32),
        pltpu.VMEM((4, D), jnp.float32),
        pltpu.VMEM((1, D // 2), jnp.int32),
        pltpu.SemaphoreType.DMA,
        pltpu.SemaphoreType.DMA,
        pltpu.SemaphoreType.DMA,
        pltpu.SemaphoreType.DMA,
        pltpu.SemaphoreType.DMA,
    ],
    compiler_params=pltpu.CompilerParams(use_tc_tiling_on_sc=False,
                                         needs_layout_passes=False),
)(_pool_body)


def _bn_body(x_ref, g_ref, b_ref, o_ref):
    x = x_ref[...]
    mu = jnp.mean(x, axis=0, keepdims=True)
    xc = x - mu
    var = jnp.mean(xc * xc, axis=0, keepdims=True)
    o_ref[...] = g_ref[...] * (xc * lax.rsqrt(var + 1e-5)) + b_ref[...]


_bn = pl.pallas_call(
    _bn_body,
    out_shape=jax.ShapeDtypeStruct((B, D), jnp.float32),
)


def kernel(title, body, emb_table, gamma, beta):
    del title
    packed = lax.bitcast_convert_type(
        emb_table.astype(jnp.bfloat16).reshape(emb_table.shape[0], D // 2, 2),
        jnp.int32)
    pooled = _pool(body.astype(jnp.int32), packed)
    return _bn(pooled, gamma.reshape(1, D), beta.reshape(1, D))


# speedup vs baseline: 2.5597x; 2.5597x over previous
import functools

import jax
import jax.numpy as jnp
from jax import lax
from jax.experimental import pallas as pl
from jax.experimental.pallas import tpu as pltpu
from jax.experimental.pallas import tpu_sc as plsc

D = 128
B = 4096
L = 200

NC = 2
NS = 16
NW = NC * NS
RPW = B // NW
LANES = 16
NG = D // 32
C0, C1 = 104, 96
HMASK = -65536

_mesh = plsc.VectorSubcoreMesh(core_axis_name="c", subcore_axis_name="s")


def _split(vb):
    v = plsc.bitcast(vb, jnp.int32)
    lo = plsc.bitcast(jnp.left_shift(v, 16), jnp.float32)
    hi = plsc.bitcast(jnp.bitwise_and(v, HMASK), jnp.float32)
    return lo, hi


def _pool_body(body_hbm, table_hbm, out_hbm, idx_v, buf0, buf1, buf2, buf3,
               out4, e0_v, sem0, sem1, sem2, sem3, outsem):
    wid = lax.axis_index("s") * NC + lax.axis_index("c")
    base = wid * RPW
    pltpu.sync_copy(body_hbm.at[pl.ds(base, RPW)], idx_v)
    pltpu.sync_copy(table_hbm.at[pl.ds(0, 1)], e0_v)

    def _issue(row, buf, sem):
        pltpu.async_copy(table_hbm.at[idx_v.at[row, pl.ds(0, C0)]],
                         buf.at[pl.ds(0, C0)], sem)
        pltpu.async_copy(table_hbm.at[idx_v.at[row, pl.ds(C0, C1)]],
                         buf.at[pl.ds(C0, C1)], sem)

    def _wait(buf, sem):
        pltpu.make_async_copy(table_hbm.at[idx_v.at[0, pl.ds(0, C0)]],
                              buf.at[pl.ds(0, C0)], sem).wait()
        pltpu.make_async_copy(table_hbm.at[idx_v.at[0, pl.ds(C0, C1)]],
                              buf.at[pl.ds(C0, C1)], sem).wait()

    lane = lax.iota(jnp.int32, LANES)
    idx_a = lax.shift_right_logical(lane, 1)
    idx_b = idx_a + 8
    even = jnp.bitwise_and(lane, 1) == 0
    e0 = [_split(e0_v[0, pl.ds(g * 32, 32)]) for g in range(NG)]
    zero = jnp.zeros((LANES,), jnp.float32)

    _dnums = lax.GatherDimensionNumbers(offset_dims=(),
                                        collapsed_slice_dims=(0,),
                                        start_index_map=(0,))

    def _take(v, i):
        return lax.gather(v, i[:, None], _dnums, slice_sizes=(1,),
                          mode=lax.GatherScatterMode.PROMISE_IN_BOUNDS)

    def _process(row, buf, slot):
        cnt = zero
        for c in range(12):
            cnt = cnt + jnp.where(idx_v[row, pl.ds(c * 16, 16)] > 0, 1.0, 0.0)
        tail = (idx_v[row, pl.ds(184, 16)] > 0) & (lane >= 8)
        cnt = cnt + jnp.where(tail, 1.0, 0.0)
        nnzf = jnp.broadcast_to(jnp.sum(cnt), (LANES,))
        n0 = float(L) - nnzf
        inv = 1.0 / jnp.maximum(nnzf, 1.0)

        def acc_step(t, accs):
            los, his = accs
            l = t * 4
            for u in range(4):
                for g in range(NG):
                    lo, hi = _split(buf[l + u, pl.ds(g * 32, 32)])
                    los = tuple(los[k] + lo if k == g else los[k]
                                for k in range(NG))
                    his = tuple(his[k] + hi if k == g else his[k]
                                for k in range(NG))
            return los, his

        los, his = lax.fori_loop(
            0, L // 4, acc_step,
            (tuple(zero for _ in range(NG)), tuple(zero for _ in range(NG))))
        for g in range(NG):
            lo = (los[g] - n0 * e0[g][0]) * inv
            hi = (his[g] - n0 * e0[g][1]) * inv
            fa = jnp.where(even, _take(lo, idx_a), _take(hi, idx_a))
            fb = jnp.where(even, _take(lo, idx_b), _take(hi, idx_b))
            out4[slot, pl.ds(g * 32, 16)] = fa
            out4[slot, pl.ds(g * 32 + 16, 16)] = fb
        pltpu.async_copy(out4.at[pl.ds(slot, 1)],
                         out_hbm.at[pl.ds(base + row, 1)], outsem)

    bufs = (buf0, buf1, buf2, buf3)
    sems = (sem0, sem1, sem2, sem3)
    for b in range(4):
        _issue(b, bufs[b], sems[b])

    def _wait_store(slot):
        pltpu.make_async_copy(out4.at[pl.ds(slot, 1)],
                              out_hbm.at[pl.ds(base, 1)], outsem).wait()

    def outer(t, carry):
        for b in range(4):
            row = 4 * t + b
            _wait(bufs[b], sems[b])

            @pl.when(row >= 4)
            def _():
                _wait_store(b)

            _process(row, bufs[b], b)

            @pl.when(row + 4 < RPW)
            def _():
                _issue(row + 4, bufs[b], sems[b])

        return carry

    lax.fori_loop(0, RPW // 4, outer, 0)
    for b in range(4):
        _wait_store(b)


_pool = functools.partial(
    pl.kernel,
    out_type=jax.ShapeDtypeStruct((B, D), jnp.float32),
    mesh=_mesh,
    scratch_types=[
        pltpu.VMEM((RPW, L), jnp.int32),
        pltpu.VMEM((L, D), jnp.bfloat16),
        pltpu.VMEM((L, D), jnp.bfloat16),
        pltpu.VMEM((L, D), jnp.bfloat16),
        pltpu.VMEM((L, D), jnp.bfloat16),
        pltpu.VMEM((4, D), jnp.float32),
        pltpu.VMEM((1, D), jnp.bfloat16),
        pltpu.SemaphoreType.DMA,
        pltpu.SemaphoreType.DMA,
        pltpu.SemaphoreType.DMA,
        pltpu.SemaphoreType.DMA,
        pltpu.SemaphoreType.DMA,
    ],
    compiler_params=pltpu.CompilerParams(use_tc_tiling_on_sc=False,
                                         needs_layout_passes=False),
)(_pool_body)


def _bn_body(x_ref, g_ref, b_ref, o_ref):
    x = x_ref[...]
    mu = jnp.mean(x, axis=0, keepdims=True)
    xc = x - mu
    var = jnp.mean(xc * xc, axis=0, keepdims=True)
    o_ref[...] = g_ref[...] * (xc * lax.rsqrt(var + 1e-5)) + b_ref[...]


_bn = pl.pallas_call(
    _bn_body,
    out_shape=jax.ShapeDtypeStruct((B, D), jnp.float32),
)


def kernel(title, body, emb_table, gamma, beta):
    del title
    pooled = _pool(body.astype(jnp.int32), emb_table.astype(jnp.bfloat16))
    return _bn(pooled, gamma.reshape(1, D), beta.reshape(1, D))


# speedup vs baseline: 2.7733x; 1.0834x over previous
import functools

import jax
import jax.numpy as jnp
from jax import lax
from jax.experimental import pallas as pl
from jax.experimental.pallas import tpu as pltpu
from jax.experimental.pallas import tpu_sc as plsc

V = 100000
D = 128
B = 4096
L = 200

NC = 2
NS = 16
NW = NC * NS
RPW = B // NW
LANES = 16
NG = D // 32
C0, C1 = 104, 96
HMASK = -65536

VPW = V // NW
CROWS = 125
NCHUNK = VPW // CROWS

_mesh = plsc.VectorSubcoreMesh(core_axis_name="c", subcore_axis_name="s")
_params = pltpu.CompilerParams(use_tc_tiling_on_sc=False,
                               needs_layout_passes=False)


def _bf16_bits(b):
    lsb = jnp.bitwise_and(lax.shift_right_logical(b, 16), 1)
    return b + lsb + 0x7FFF


def _convert_body(tbl_hbm, out_hbm, vin0, vin1, vout0, vout1,
                  isem0, isem1, osem0, osem1):
    wid = lax.axis_index("s") * NC + lax.axis_index("c")
    rbase = wid * VPW

    def _issue_in(c, vin, isem):
        pltpu.async_copy(tbl_hbm.at[pl.ds(rbase + c * CROWS, CROWS)], vin,
                         isem)

    def _wait_in(vin, isem):
        pltpu.make_async_copy(tbl_hbm.at[pl.ds(0, CROWS)], vin, isem).wait()

    def _store_out(c, vout, osem):
        pltpu.async_copy(vout, out_hbm.at[pl.ds(rbase + c * CROWS, CROWS)],
                         osem)

    def _wait_out(vout, osem):
        pltpu.make_async_copy(vout, out_hbm.at[pl.ds(0, CROWS)], osem).wait()

    def _compute(vin, vout):
        def row(r, carry):
            for g in range(NG):
                b0 = plsc.bitcast(vin[r, pl.ds(g * 32, LANES)], jnp.int32)
                b1 = plsc.bitcast(vin[r, pl.ds(g * 32 + 16, LANES)],
                                  jnp.int32)
                lo = lax.shift_right_logical(_bf16_bits(b0), 16)
                hi = jnp.bitwise_and(_bf16_bits(b1), HMASK)
                vout[r, pl.ds(g * LANES, LANES)] = jnp.bitwise_or(lo, hi)
            return carry

        lax.fori_loop(0, CROWS, row, 0)

    bufs = ((vin0, isem0, vout0, osem0), (vin1, isem1, vout1, osem1))
    _issue_in(0, vin0, isem0)
    _issue_in(1, vin1, isem1)

    def loop(t, carry):
        for p in range(2):
            c = 2 * t + p
            vin, isem, vout, osem = bufs[p]
            _wait_in(vin, isem)

            @pl.when(t > 0)
            def _():
                _wait_out(vout, osem)

            _compute(vin, vout)
            _store_out(c, vout, osem)

            @pl.when(c + 2 < NCHUNK)
            def _():
                _issue_in(c + 2, vin, isem)

        return carry

    lax.fori_loop(0, (NCHUNK - 1) // 2, loop, 0)
    vin, isem, vout, osem = bufs[0]
    _wait_in(vin, isem)
    _wait_out(vout, osem)
    _compute(vin, vout)
    _store_out(NCHUNK - 1, vout, osem)
    _wait_out(vout0, osem0)
    _wait_out(vout1, osem1)


_convert = functools.partial(
    pl.kernel,
    out_type=jax.ShapeDtypeStruct((V, D // 2), jnp.int32),
    mesh=_mesh,
    scratch_types=[
        pltpu.VMEM((CROWS, D), jnp.float32),
        pltpu.VMEM((CROWS, D), jnp.float32),
        pltpu.VMEM((CROWS, D // 2), jnp.int32),
        pltpu.VMEM((CROWS, D // 2), jnp.int32),
        pltpu.SemaphoreType.DMA,
        pltpu.SemaphoreType.DMA,
        pltpu.SemaphoreType.DMA,
        pltpu.SemaphoreType.DMA,
    ],
    compiler_params=_params,
)(_convert_body)


def _split(v):
    lo = plsc.bitcast(jnp.left_shift(v, 16), jnp.float32)
    hi = plsc.bitcast(jnp.bitwise_and(v, HMASK), jnp.float32)
    return lo, hi


def _pool_body(body_hbm, table_hbm, out_hbm, idx_v, buf0, buf1, buf2, buf3,
               out4, e0_v, sem0, sem1, sem2, sem3, outsem):
    wid = lax.axis_index("s") * NC + lax.axis_index("c")
    base = wid * RPW
    pltpu.sync_copy(body_hbm.at[pl.ds(base, RPW)], idx_v)
    pltpu.sync_copy(table_hbm.at[pl.ds(0, 1)], e0_v)

    def _issue(row, buf, sem):
        pltpu.async_copy(table_hbm.at[idx_v.at[row, pl.ds(0, C0)]],
                         buf.at[pl.ds(0, C0)], sem)
        pltpu.async_copy(table_hbm.at[idx_v.at[row, pl.ds(C0, C1)]],
                         buf.at[pl.ds(C0, C1)], sem)

    def _wait(buf, sem):
        pltpu.make_async_copy(table_hbm.at[idx_v.at[0, pl.ds(0, C0)]],
                              buf.at[pl.ds(0, C0)], sem).wait()
        pltpu.make_async_copy(table_hbm.at[idx_v.at[0, pl.ds(C0, C1)]],
                              buf.at[pl.ds(C0, C1)], sem).wait()

    lane = lax.iota(jnp.int32, LANES)
    e0 = [_split(e0_v[0, pl.ds(g * LANES, LANES)]) for g in range(NG)]
    zero = jnp.zeros((LANES,), jnp.float32)

    def _process(row, buf, slot):
        cnt = zero
        for c in range(12):
            cnt = cnt + jnp.where(idx_v[row, pl.ds(c * 16, 16)] > 0, 1.0, 0.0)
        tail = (idx_v[row, pl.ds(184, 16)] > 0) & (lane >= 8)
        cnt = cnt + jnp.where(tail, 1.0, 0.0)
        nnzf = jnp.broadcast_to(jnp.sum(cnt), (LANES,))
        n0 = float(L) - nnzf
        inv = 1.0 / jnp.maximum(nnzf, 1.0)

        def acc_step(t, accs):
            los, his = accs
            l = t * 4
            for u in range(4):
                for g in range(NG):
                    lo, hi = _split(buf[l + u, pl.ds(g * LANES, LANES)])
                    los = tuple(los[k] + lo if k == g else los[k]
                                for k in range(NG))
                    his = tuple(his[k] + hi if k == g else his[k]
                                for k in range(NG))
            return los, his

        los, his = lax.fori_loop(
            0, L // 4, acc_step,
            (tuple(zero for _ in range(NG)), tuple(zero for _ in range(NG))))
        for g in range(NG):
            out4[slot, pl.ds(g * 32, 16)] = (los[g] - n0 * e0[g][0]) * inv
            out4[slot, pl.ds(g * 32 + 16, 16)] = (his[g] - n0 * e0[g][1]) * inv
        pltpu.async_copy(out4.at[pl.ds(slot, 1)],
                         out_hbm.at[pl.ds(base + row, 1)], outsem)

    bufs = (buf0, buf1, buf2, buf3)
    sems = (sem0, sem1, sem2, sem3)
    for b in range(4):
        _issue(b, bufs[b], sems[b])

    def _wait_store(slot):
        pltpu.make_async_copy(out4.at[pl.ds(slot, 1)],
                              out_hbm.at[pl.ds(base, 1)], outsem).wait()

    def outer(t, carry):
        for b in range(4):
            row = 4 * t + b
            _wait(bufs[b], sems[b])

            @pl.when(row >= 4)
            def _():
                _wait_store(b)

            _process(row, bufs[b], b)

            @pl.when(row + 4 < RPW)
            def _():
                _issue(row + 4, bufs[b], sems[b])

        return carry

    lax.fori_loop(0, RPW // 4, outer, 0)
    for b in range(4):
        _wait_store(b)


_pool = functools.partial(
    pl.kernel,
    out_type=jax.ShapeDtypeStruct((B, D), jnp.float32),
    mesh=_mesh,
    scratch_types=[
        pltpu.VMEM((RPW, L), jnp.int32),
        pltpu.VMEM((L, D // 2), jnp.int32),
        pltpu.VMEM((L, D // 2), jnp.int32),
        pltpu.VMEM((L, D // 2), jnp.int32),
        pltpu.VMEM((L, D // 2), jnp.int32),
        pltpu.VMEM((4, D), jnp.float32),
        pltpu.VMEM((1, D // 2), jnp.int32),
        pltpu.SemaphoreType.DMA,
        pltpu.SemaphoreType.DMA,
        pltpu.SemaphoreType.DMA,
        pltpu.SemaphoreType.DMA,
        pltpu.SemaphoreType.DMA,
    ],
    compiler_params=_params,
)(_pool_body)


def _bn_body(x_ref, g_ref, b_ref, o_ref):
    x = x_ref[...]
    mu = jnp.mean(x, axis=0, keepdims=True)
    xc = x - mu
    var = jnp.mean(xc * xc, axis=0, keepdims=True)
    o_ref[...] = g_ref[...] * (xc * lax.rsqrt(var + 1e-5)) + b_ref[...]


_bn = pl.pallas_call(
    _bn_body,
    out_shape=jax.ShapeDtypeStruct((B, D), jnp.float32),
)


def kernel(title, body, emb_table, gamma, beta):
    del title
    packed = _convert(emb_table)
    pooled = _pool(body.astype(jnp.int32), packed)
    return _bn(pooled, gamma.reshape(1, D), beta.reshape(1, D))


# speedup vs baseline: 3.0355x; 1.0946x over previous
import functools

import jax
import jax.numpy as jnp
from jax import lax
from jax.experimental import pallas as pl
from jax.experimental.pallas import tpu as pltpu
from jax.experimental.pallas import tpu_sc as plsc

V = 100000
D = 128
B = 4096
L = 200

NC = 2
NS = 16
NW = NC * NS
RPW = B // NW
LANES = 16
NG = D // 32
C0, C1 = 104, 96
HMASK = -65536

VPW = V // NW
CROWS = 125
NCHUNK = VPW // CROWS

_mesh = plsc.VectorSubcoreMesh(core_axis_name="c", subcore_axis_name="s")
_params = pltpu.CompilerParams(use_tc_tiling_on_sc=False,
                               needs_layout_passes=False)


def _convert_body(tbl_hbm, out_hbm, vin0, vin1, vin2, vin3,
                  vout0, vout1, vout2, vout3,
                  isem0, isem1, isem2, isem3, osem0, osem1, osem2, osem3):
    wid = lax.axis_index("s") * NC + lax.axis_index("c")
    rbase = wid * VPW

    def _issue_in(c, vin, isem):
        pltpu.async_copy(tbl_hbm.at[pl.ds(rbase + c * CROWS, CROWS)], vin,
                         isem)

    def _wait_in(vin, isem):
        pltpu.make_async_copy(tbl_hbm.at[pl.ds(0, CROWS)], vin, isem).wait()

    def _store_out(c, vout, osem):
        pltpu.async_copy(vout, out_hbm.at[pl.ds(rbase + c * CROWS, CROWS)],
                         osem)

    def _wait_out(vout, osem):
        pltpu.make_async_copy(vout, out_hbm.at[pl.ds(0, CROWS)], osem).wait()

    def _compute(vin, vout):
        def row(rr, carry):
            r = rr * 5
            for q in range(5):
                for g in range(NG):
                    b0 = plsc.bitcast(vin[r + q, pl.ds(g * 32, LANES)],
                                      jnp.int32) + 0x8000
                    b1 = plsc.bitcast(vin[r + q, pl.ds(g * 32 + 16, LANES)],
                                      jnp.int32) + 0x8000
                    lo = lax.shift_right_logical(b0, 16)
                    hi = jnp.bitwise_and(b1, HMASK)
                    vout[r + q, pl.ds(g * LANES, LANES)] = jnp.bitwise_or(
                        lo, hi)
            return carry

        lax.fori_loop(0, CROWS // 5, row, 0)

    bufs = ((vin0, isem0, vout0, osem0), (vin1, isem1, vout1, osem1),
            (vin2, isem2, vout2, osem2), (vin3, isem3, vout3, osem3))
    for p in range(4):
        _issue_in(p, bufs[p][0], bufs[p][1])

    def loop(t, carry):
        for p in range(4):
            c = 4 * t + p
            vin, isem, vout, osem = bufs[p]
            _wait_in(vin, isem)

            @pl.when(c >= 4)
            def _():
                _wait_out(vout, osem)

            _compute(vin, vout)
            _store_out(c, vout, osem)

            @pl.when(c + 4 < NCHUNK)
            def _():
                _issue_in(c + 4, vin, isem)

        return carry

    lax.fori_loop(0, NCHUNK // 4, loop, 0)
    vin, isem, vout, osem = bufs[0]
    _wait_in(vin, isem)
    _wait_out(vout, osem)
    _compute(vin, vout)
    _store_out(NCHUNK - 1, vout, osem)
    for p in range(4):
        _wait_out(bufs[p][2], bufs[p][3])


_convert = functools.partial(
    pl.kernel,
    out_type=jax.ShapeDtypeStruct((V, D // 2), jnp.int32),
    mesh=_mesh,
    scratch_types=(
        [pltpu.VMEM((CROWS, D), jnp.float32)] * 4
        + [pltpu.VMEM((CROWS, D // 2), jnp.int32)] * 4
        + [pltpu.SemaphoreType.DMA] * 8
    ),
    compiler_params=_params,
)(_convert_body)


def _split(v):
    lo = plsc.bitcast(jnp.left_shift(v, 16), jnp.float32)
    hi = plsc.bitcast(jnp.bitwise_and(v, HMASK), jnp.float32)
    return lo, hi


def _pool_body(body_hbm, table_hbm, out_hbm, idx_v, buf0, buf1, buf2, buf3,
               out4, e0_v, sem0, sem1, sem2, sem3, outsem):
    wid = lax.axis_index("s") * NC + lax.axis_index("c")
    base = wid * RPW
    pltpu.sync_copy(body_hbm.at[pl.ds(base, RPW)], idx_v)
    pltpu.sync_copy(table_hbm.at[pl.ds(0, 1)], e0_v)

    def _issue(row, buf, sem):
        pltpu.async_copy(table_hbm.at[idx_v.at[row, pl.ds(0, C0)]],
                         buf.at[pl.ds(0, C0)], sem)
        pltpu.async_copy(table_hbm.at[idx_v.at[row, pl.ds(C0, C1)]],
                         buf.at[pl.ds(C0, C1)], sem)

    def _wait(buf, sem):
        pltpu.make_async_copy(table_hbm.at[idx_v.at[0, pl.ds(0, C0)]],
                              buf.at[pl.ds(0, C0)], sem).wait()
        pltpu.make_async_copy(table_hbm.at[idx_v.at[0, pl.ds(C0, C1)]],
                              buf.at[pl.ds(C0, C1)], sem).wait()

    lane = lax.iota(jnp.int32, LANES)
    e0 = [_split(e0_v[0, pl.ds(g * LANES, LANES)]) for g in range(NG)]
    zero = jnp.zeros((LANES,), jnp.float32)

    def _process(row, buf, slot):
        cnt = zero
        for c in range(12):
            cnt = cnt + jnp.where(idx_v[row, pl.ds(c * 16, 16)] > 0, 1.0, 0.0)
        tail = (idx_v[row, pl.ds(184, 16)] > 0) & (lane >= 8)
        cnt = cnt + jnp.where(tail, 1.0, 0.0)
        nnzf = jnp.broadcast_to(jnp.sum(cnt), (LANES,))
        n0 = float(L) - nnzf
        inv = 1.0 / jnp.maximum(nnzf, 1.0)

        def acc_step(t, accs):
            los, his = accs
            l = t * 4
            for u in range(4):
                for g in range(NG):
                    lo, hi = _split(buf[l + u, pl.ds(g * LANES, LANES)])
                    los = tuple(los[k] + lo if k == g else los[k]
                                for k in range(NG))
                    his = tuple(his[k] + hi if k == g else his[k]
                                for k in range(NG))
            return los, his

        los, his = lax.fori_loop(
            0, L // 4, acc_step,
            (tuple(zero for _ in range(NG)), tuple(zero for _ in range(NG))))
        for g in range(NG):
            out4[slot, pl.ds(g * 32, 16)] = (los[g] - n0 * e0[g][0]) * inv
            out4[slot, pl.ds(g * 32 + 16, 16)] = (his[g] - n0 * e0[g][1]) * inv
        pltpu.async_copy(out4.at[pl.ds(slot, 1)],
                         out_hbm.at[pl.ds(base + row, 1)], outsem)

    bufs = (buf0, buf1, buf2, buf3)
    sems = (sem0, sem1, sem2, sem3)
    for b in range(4):
        _issue(b, bufs[b], sems[b])

    def _wait_store(slot):
        pltpu.make_async_copy(out4.at[pl.ds(slot, 1)],
                              out_hbm.at[pl.ds(base, 1)], outsem).wait()

    def outer(t, carry):
        for b in range(4):
            row = 4 * t + b
            _wait(bufs[b], sems[b])

            @pl.when(row >= 4)
            def _():
                _wait_store(b)

            _process(row, bufs[b], b)

            @pl.when(row + 4 < RPW)
            def _():
                _issue(row + 4, bufs[b], sems[b])

        return carry

    lax.fori_loop(0, RPW // 4, outer, 0)
    for b in range(4):
        _wait_store(b)


_pool = functools.partial(
    pl.kernel,
    out_type=jax.ShapeDtypeStruct((B, D), jnp.float32),
    mesh=_mesh,
    scratch_types=[
        pltpu.VMEM((RPW, L), jnp.int32),
        pltpu.VMEM((L, D // 2), jnp.int32),
        pltpu.VMEM((L, D // 2), jnp.int32),
        pltpu.VMEM((L, D // 2), jnp.int32),
        pltpu.VMEM((L, D // 2), jnp.int32),
        pltpu.VMEM((4, D), jnp.float32),
        pltpu.VMEM((1, D // 2), jnp.int32),
        pltpu.SemaphoreType.DMA,
        pltpu.SemaphoreType.DMA,
        pltpu.SemaphoreType.DMA,
        pltpu.SemaphoreType.DMA,
        pltpu.SemaphoreType.DMA,
    ],
    compiler_params=_params,
)(_pool_body)


def _bn_body(x_ref, g_ref, b_ref, o_ref):
    x = x_ref[...]
    mu = jnp.mean(x, axis=0, keepdims=True)
    xc = x - mu
    var = jnp.mean(xc * xc, axis=0, keepdims=True)
    o_ref[...] = g_ref[...] * (xc * lax.rsqrt(var + 1e-5)) + b_ref[...]


_bn = pl.pallas_call(
    _bn_body,
    out_shape=jax.ShapeDtypeStruct((B, D), jnp.float32),
)


def kernel(title, body, emb_table, gamma, beta):
    del title
    packed = _convert(emb_table)
    pooled = _pool(body.astype(jnp.int32), packed)
    return _bn(pooled, gamma.reshape(1, D), beta.reshape(1, D))


# speedup vs baseline: 3.9219x; 1.2920x over previous
import functools

import jax
import jax.numpy as jnp
from jax import lax
from jax.experimental import pallas as pl
from jax.experimental.pallas import tpu as pltpu
from jax.experimental.pallas import tpu_sc as plsc

V = 100000
D = 128
B = 4096
L = 200

NC = 2
NS = 16
NW = NC * NS
RPW = B // NW
LANES = 16
NG = D // 32
C0, C1 = 104, 96
HMASK = -65536

VPW = V // NW
CROWS = 125
NCHUNK = VPW // CROWS

_mesh = plsc.VectorSubcoreMesh(core_axis_name="c", subcore_axis_name="s")
_params = pltpu.CompilerParams(use_tc_tiling_on_sc=False,
                               needs_layout_passes=False)


def _convert_body(tbl_hbm, out_hbm, vin0, vin1, vin2, vin3,
                  vout0, vout1, vout2, vout3,
                  isem0, isem1, isem2, isem3, osem0, osem1, osem2, osem3):
    wid = lax.axis_index("s") * NC + lax.axis_index("c")
    rbase = wid * VPW

    def _issue_in(c, vin, isem):
        pltpu.async_copy(tbl_hbm.at[pl.ds(rbase + c * CROWS, CROWS)], vin,
                         isem)

    def _wait_in(vin, isem):
        pltpu.make_async_copy(tbl_hbm.at[pl.ds(0, CROWS)], vin, isem).wait()

    def _store_out(c, vout, osem):
        pltpu.async_copy(vout, out_hbm.at[pl.ds(rbase + c * CROWS, CROWS)],
                         osem)

    def _wait_out(vout, osem):
        pltpu.make_async_copy(vout, out_hbm.at[pl.ds(0, CROWS)], osem).wait()

    def _compute(vin, vout):
        @plsc.parallel_loop(0, CROWS, 5, unroll=2)
        def _rows(r):
            for q in range(5):
                for g in range(NG):
                    b0 = plsc.bitcast(vin[r + q, pl.ds(g * 32, LANES)],
                                      jnp.int32) + 0x8000
                    b1 = plsc.bitcast(vin[r + q, pl.ds(g * 32 + 16, LANES)],
                                      jnp.int32) + 0x8000
                    lo = lax.shift_right_logical(b0, 16)
                    hi = jnp.bitwise_and(b1, HMASK)
                    vout[r + q, pl.ds(g * LANES, LANES)] = jnp.bitwise_or(
                        lo, hi)

    bufs = ((vin0, isem0, vout0, osem0), (vin1, isem1, vout1, osem1),
            (vin2, isem2, vout2, osem2), (vin3, isem3, vout3, osem3))
    for p in range(4):
        _issue_in(p, bufs[p][0], bufs[p][1])

    def loop(t, carry):
        for p in range(4):
            c = 4 * t + p
            vin, isem, vout, osem = bufs[p]
            _wait_in(vin, isem)

            @pl.when(c >= 4)
            def _():
                _wait_out(vout, osem)

            _compute(vin, vout)
            _store_out(c, vout, osem)

            @pl.when(c + 4 < NCHUNK)
            def _():
                _issue_in(c + 4, vin, isem)

        return carry

    lax.fori_loop(0, NCHUNK // 4, loop, 0)
    vin, isem, vout, osem = bufs[0]
    _wait_in(vin, isem)
    _wait_out(vout, osem)
    _compute(vin, vout)
    _store_out(NCHUNK - 1, vout, osem)
    for p in range(4):
        _wait_out(bufs[p][2], bufs[p][3])


_convert = functools.partial(
    pl.kernel,
    out_type=jax.ShapeDtypeStruct((V, D // 2), jnp.int32),
    mesh=_mesh,
    scratch_types=(
        [pltpu.VMEM((CROWS, D), jnp.float32)] * 4
        + [pltpu.VMEM((CROWS, D // 2), jnp.int32)] * 4
        + [pltpu.SemaphoreType.DMA] * 8
    ),
    compiler_params=_params,
)(_convert_body)


def _split(v):
    lo = plsc.bitcast(jnp.left_shift(v, 16), jnp.float32)
    hi = plsc.bitcast(jnp.bitwise_and(v, HMASK), jnp.float32)
    return lo, hi


def _pool_body(body_hbm, table_hbm, out_hbm, idx_v, buf0, buf1, buf2, buf3,
               out4, e0_v, sem0, sem1, sem2, sem3, outsem):
    wid = lax.axis_index("s") * NC + lax.axis_index("c")
    base = wid * RPW
    pltpu.sync_copy(body_hbm.at[pl.ds(base, RPW)], idx_v)
    pltpu.sync_copy(table_hbm.at[pl.ds(0, 1)], e0_v)

    def _issue(row, buf, sem):
        pltpu.async_copy(table_hbm.at[idx_v.at[row, pl.ds(0, C0)]],
                         buf.at[pl.ds(0, C0)], sem)
        pltpu.async_copy(table_hbm.at[idx_v.at[row, pl.ds(C0, C1)]],
                         buf.at[pl.ds(C0, C1)], sem)

    def _wait(buf, sem):
        pltpu.make_async_copy(table_hbm.at[idx_v.at[0, pl.ds(0, C0)]],
                              buf.at[pl.ds(0, C0)], sem).wait()
        pltpu.make_async_copy(table_hbm.at[idx_v.at[0, pl.ds(C0, C1)]],
                              buf.at[pl.ds(C0, C1)], sem).wait()

    lane = lax.iota(jnp.int32, LANES)
    e0 = [_split(e0_v[0, pl.ds(g * LANES, LANES)]) for g in range(NG)]
    zero = jnp.zeros((LANES,), jnp.float32)

    def _process(row, buf, slot):
        cnt = zero
        for c in range(12):
            cnt = cnt + jnp.where(idx_v[row, pl.ds(c * 16, 16)] > 0, 1.0, 0.0)
        tail = (idx_v[row, pl.ds(184, 16)] > 0) & (lane >= 8)
        cnt = cnt + jnp.where(tail, 1.0, 0.0)
        nnzf = jnp.broadcast_to(jnp.sum(cnt), (LANES,))
        n0 = float(L) - nnzf
        inv = 1.0 / jnp.maximum(nnzf, 1.0)

        def acc_step(t, accs):
            los, his = accs
            l = t * 4
            for u in range(4):
                for g in range(NG):
                    lo, hi = _split(buf[l + u, pl.ds(g * LANES, LANES)])
                    los = tuple(los[k] + lo if k == g else los[k]
                                for k in range(NG))
                    his = tuple(his[k] + hi if k == g else his[k]
                                for k in range(NG))
            return los, his

        los, his = lax.fori_loop(
            0, L // 4, acc_step,
            (tuple(zero for _ in range(NG)), tuple(zero for _ in range(NG))))
        for g in range(NG):
            out4[slot, pl.ds(g * 32, 16)] = (los[g] - n0 * e0[g][0]) * inv
            out4[slot, pl.ds(g * 32 + 16, 16)] = (his[g] - n0 * e0[g][1]) * inv
        pltpu.async_copy(out4.at[pl.ds(slot, 1)],
                         out_hbm.at[pl.ds(base + row, 1)], outsem)

    bufs = (buf0, buf1, buf2, buf3)
    sems = (sem0, sem1, sem2, sem3)
    for b in range(4):
        _issue(b, bufs[b], sems[b])

    def _wait_store(slot):
        pltpu.make_async_copy(out4.at[pl.ds(slot, 1)],
                              out_hbm.at[pl.ds(base, 1)], outsem).wait()

    def outer(t, carry):
        for b in range(4):
            row = 4 * t + b
            _wait(bufs[b], sems[b])

            @pl.when(row >= 4)
            def _():
                _wait_store(b)

            _process(row, bufs[b], b)

            @pl.when(row + 4 < RPW)
            def _():
                _issue(row + 4, bufs[b], sems[b])

        return carry

    lax.fori_loop(0, RPW // 4, outer, 0)
    for b in range(4):
        _wait_store(b)


_pool = functools.partial(
    pl.kernel,
    out_type=jax.ShapeDtypeStruct((B, D), jnp.float32),
    mesh=_mesh,
    scratch_types=[
        pltpu.VMEM((RPW, L), jnp.int32),
        pltpu.VMEM((L, D // 2), jnp.int32),
        pltpu.VMEM((L, D // 2), jnp.int32),
        pltpu.VMEM((L, D // 2), jnp.int32),
        pltpu.VMEM((L, D // 2), jnp.int32),
        pltpu.VMEM((4, D), jnp.float32),
        pltpu.VMEM((1, D // 2), jnp.int32),
        pltpu.SemaphoreType.DMA,
        pltpu.SemaphoreType.DMA,
        pltpu.SemaphoreType.DMA,
        pltpu.SemaphoreType.DMA,
        pltpu.SemaphoreType.DMA,
    ],
    compiler_params=_params,
)(_pool_body)


def _bn_body(x_ref, g_ref, b_ref, o_ref):
    x = x_ref[...]
    mu = jnp.mean(x, axis=0, keepdims=True)
    xc = x - mu
    var = jnp.mean(xc * xc, axis=0, keepdims=True)
    o_ref[...] = g_ref[...] * (xc * lax.rsqrt(var + 1e-5)) + b_ref[...]


_bn = pl.pallas_call(
    _bn_body,
    out_shape=jax.ShapeDtypeStruct((B, D), jnp.float32),
)


def kernel(title, body, emb_table, gamma, beta):
    del title
    packed = _convert(emb_table)
    pooled = _pool(body.astype(jnp.int32), packed)
    return _bn(pooled, gamma.reshape(1, D), beta.reshape(1, D))
